# baseline (speedup 1.0000x reference)
import math

import jax
import jax.numpy as jnp
from jax import lax
from jax.experimental import pallas as pl
from jax.experimental.pallas import tpu as pltpu
from jax.experimental.pallas import tpu_sc as plsc

D_MODEL = 512
SCALE = math.sqrt(D_MODEL)

NUM_CORES = 2
NUM_SUBCORES = 16
NUM_LANES = 16
NW = NUM_CORES * NUM_SUBCORES

CHUNK = 64


def _sc_embedding(idx3, table, n_chunks):
    b_total = NW * n_chunks * CHUNK
    mesh = plsc.VectorSubcoreMesh(core_axis_name="c", subcore_axis_name="s")

    def body(idx_hbm, table_hbm, out_hbm, idx_v, buf, sem):
        wid = lax.axis_index("s") * NUM_CORES + lax.axis_index("c")
        pltpu.sync_copy(idx_hbm.at[wid], idx_v)
        base = wid * (n_chunks * CHUNK)

        def chunk_body(c, _):
            pltpu.async_copy(table_hbm.at[idx_v.at[c]], buf, sem).wait()

            def scale_row(r, _):
                for g in range(D_MODEL // NUM_LANES):
                    sl = pl.ds(g * NUM_LANES, NUM_LANES)
                    buf[r, sl] = buf[r, sl] * SCALE
                return 0

            lax.fori_loop(0, CHUNK, scale_row, 0)
            pltpu.sync_copy(buf, out_hbm.at[pl.ds(base + c * CHUNK, CHUNK)])
            return 0

        lax.fori_loop(0, n_chunks, chunk_body, 0)

    run = pl.kernel(
        body,
        out_type=jax.ShapeDtypeStruct((b_total, D_MODEL), jnp.float32),
        mesh=mesh,
        scratch_types=[
            pltpu.VMEM((n_chunks, CHUNK), jnp.int32),
            pltpu.VMEM((CHUNK, D_MODEL), jnp.float32),
            pltpu.SemaphoreType.DMA,
        ],
    )
    return run(idx3, table)


def kernel(x, table):
    b, t = x.shape
    n_total = b * t
    assert n_total % (NW * CHUNK) == 0
    n_chunks = n_total // (NW * CHUNK)
    idx3 = x.astype(jnp.int32).reshape(NW, n_chunks, CHUNK)
    out = _sc_embedding(idx3, table, n_chunks)
    return out.reshape(b, t, D_MODEL)


# speedup vs baseline: 1.1597x; 1.1597x over previous
import math

import jax
import jax.numpy as jnp
from jax import lax
from jax.experimental import pallas as pl
from jax.experimental.pallas import tpu as pltpu
from jax.experimental.pallas import tpu_sc as plsc

D_MODEL = 512
SCALE = math.sqrt(D_MODEL)

NUM_CORES = 2
NUM_SUBCORES = 16
NUM_LANES = 16
NW = NUM_CORES * NUM_SUBCORES

CHUNK = 80


def _sc_embedding(idx3, table, n_chunks):
    b_total = NW * n_chunks * CHUNK
    b_per_w = n_chunks * CHUNK
    mesh = plsc.VectorSubcoreMesh(core_axis_name="c", subcore_axis_name="s")

    def body(idx_hbm, table_hbm, out_hbm, idx_v, buf0, buf1, sem0, sem1):
        wid = lax.axis_index("s") * NUM_CORES + lax.axis_index("c")
        pltpu.sync_copy(idx_hbm.at[wid], idx_v)
        base = wid * b_per_w
        last = n_chunks - 1

        def start(c, buf, sem):
            pltpu.make_async_copy(table_hbm.at[idx_v.at[c]], buf, sem).start()

        def wait(buf, sem):
            pltpu.make_async_copy(table_hbm.at[idx_v.at[0]], buf, sem).wait()

        def scale_store(c, buf):
            def scale_row(r, _):
                for g in range(D_MODEL // NUM_LANES):
                    sl = pl.ds(g * NUM_LANES, NUM_LANES)
                    buf[r, sl] = buf[r, sl] * SCALE
                return 0

            lax.fori_loop(0, CHUNK, scale_row, 0)
            pltpu.sync_copy(buf, out_hbm.at[pl.ds(base + c * CHUNK, CHUNK)])

        start(0, buf0, sem0)

        def ring(i, _):
            c0 = 2 * i
            c1 = c0 + 1
            start(c1, buf1, sem1)
            wait(buf0, sem0)
            scale_store(c0, buf0)
            start(jnp.minimum(c1 + 1, last), buf0, sem0)
            wait(buf1, sem1)
            scale_store(c1, buf1)
            return 0

        lax.fori_loop(0, n_chunks // 2, ring, 0)
        wait(buf0, sem0)

    run = pl.kernel(
        body,
        out_type=jax.ShapeDtypeStruct((b_total, D_MODEL), jnp.float32),
        mesh=mesh,
        scratch_types=[
            pltpu.VMEM((n_chunks, CHUNK), jnp.int32),
            pltpu.VMEM((CHUNK, D_MODEL), jnp.float32),
            pltpu.VMEM((CHUNK, D_MODEL), jnp.float32),
            pltpu.SemaphoreType.DMA,
            pltpu.SemaphoreType.DMA,
        ],
    )
    return run(idx3, table)


def kernel(x, table):
    b, t = x.shape
    n_total = b * t
    assert n_total % (NW * CHUNK) == 0
    n_chunks = n_total // (NW * CHUNK)
    idx3 = x.astype(jnp.int32).reshape(NW, n_chunks, CHUNK)
    out = _sc_embedding(idx3, table, n_chunks)
    return out.reshape(b, t, D_MODEL)
